# baseline (speedup 1.0000x reference)
import jax
import jax.numpy as jnp
from jax.experimental import pallas as pl
from jax.experimental.pallas import tpu as pltpu

_G = 16
_W = 1024


def _copy_body(x_ref, r_ref, d_ref, xo_ref, ro_ref, do_ref):
    xo_ref[...] = x_ref[...]
    ro_ref[...] = r_ref[...]
    do_ref[...] = d_ref[...]


def kernel(sampled_point_xyz, sampled_point_ray_direction, sampled_point_distance):
    n = sampled_point_xyz.shape[0]
    rows3 = n * 3 // _W
    rows1 = n // _W
    b3 = rows3 // _G
    b1 = rows1 // _G
    x2 = sampled_point_xyz.reshape(rows3, _W)
    r2 = sampled_point_ray_direction.reshape(rows3, _W)
    d2 = sampled_point_distance.reshape(rows1, _W)
    pos, ray, dists = pl.pallas_call(
        _copy_body,
        grid=(_G,),
        in_specs=[
            pl.BlockSpec((b3, _W), lambda i: (i, 0)),
            pl.BlockSpec((b3, _W), lambda i: (i, 0)),
            pl.BlockSpec((b1, _W), lambda i: (i, 0)),
        ],
        out_specs=[
            pl.BlockSpec((b3, _W), lambda i: (i, 0)),
            pl.BlockSpec((b3, _W), lambda i: (i, 0)),
            pl.BlockSpec((b1, _W), lambda i: (i, 0)),
        ],
        out_shape=[
            jax.ShapeDtypeStruct((rows3, _W), jnp.float32),
            jax.ShapeDtypeStruct((rows3, _W), jnp.float32),
            jax.ShapeDtypeStruct((rows1, _W), jnp.float32),
        ],
    )(x2, r2, d2)
    return (pos.reshape(n, 3), ray.reshape(n, 3), dists.reshape(n))


# speedup vs baseline: 2.2981x; 2.2981x over previous
import jax
import jax.numpy as jnp
from jax.experimental import pallas as pl
from jax.experimental.pallas import tpu as pltpu

_G = 512


def _copy_body(x_ref, r_ref, d_ref, xo_ref, ro_ref, do_ref):
    xo_ref[...] = x_ref[...]
    ro_ref[...] = r_ref[...]
    do_ref[...] = d_ref[...]


def kernel(sampled_point_xyz, sampled_point_ray_direction, sampled_point_distance):
    n = sampled_point_xyz.shape[0]
    b = n // _G
    pos, ray, dists = pl.pallas_call(
        _copy_body,
        grid=(_G,),
        in_specs=[
            pl.BlockSpec((b, 3), lambda i: (i, 0)),
            pl.BlockSpec((b, 3), lambda i: (i, 0)),
            pl.BlockSpec((b,), lambda i: (i,)),
        ],
        out_specs=[
            pl.BlockSpec((b, 3), lambda i: (i, 0)),
            pl.BlockSpec((b, 3), lambda i: (i, 0)),
            pl.BlockSpec((b,), lambda i: (i,)),
        ],
        out_shape=[
            jax.ShapeDtypeStruct((n, 3), jnp.float32),
            jax.ShapeDtypeStruct((n, 3), jnp.float32),
            jax.ShapeDtypeStruct((n,), jnp.float32),
        ],
    )(sampled_point_xyz, sampled_point_ray_direction, sampled_point_distance)
    return (pos, ray, dists)


# speedup vs baseline: 150.4845x; 65.4814x over previous
import jax
import jax.numpy as jnp
from jax.experimental import pallas as pl
from jax.experimental.pallas import tpu as pltpu

_G = 64


def _copy_body(x_ref, r_ref, d_ref, xo_ref, ro_ref, do_ref):
    xo_ref[...] = x_ref[...]
    ro_ref[...] = r_ref[...]
    do_ref[...] = d_ref[...]


def kernel(sampled_point_xyz, sampled_point_ray_direction, sampled_point_distance):
    n = sampled_point_xyz.shape[0]
    b = n // _G
    xt = sampled_point_xyz.T
    rt = sampled_point_ray_direction.T
    pos_t, ray_t, dists = pl.pallas_call(
        _copy_body,
        grid=(_G,),
        in_specs=[
            pl.BlockSpec((3, b), lambda i: (0, i)),
            pl.BlockSpec((3, b), lambda i: (0, i)),
            pl.BlockSpec((b,), lambda i: (i,)),
        ],
        out_specs=[
            pl.BlockSpec((3, b), lambda i: (0, i)),
            pl.BlockSpec((3, b), lambda i: (0, i)),
            pl.BlockSpec((b,), lambda i: (i,)),
        ],
        out_shape=[
            jax.ShapeDtypeStruct((3, n), jnp.float32),
            jax.ShapeDtypeStruct((3, n), jnp.float32),
            jax.ShapeDtypeStruct((n,), jnp.float32),
        ],
    )(xt, rt, sampled_point_distance)
    return (pos_t.T, ray_t.T, dists)
